# Initial kernel scaffold; baseline (speedup 1.0000x reference)
import jax
import jax.numpy as jnp
from jax.experimental import pallas as pl


def kernel(x, edge_index, W1, b1, W2, b2, W3, b3):
    raise NotImplementedError("write your pallas kernel here")


# speedup vs baseline: 12.0791x; 12.0791x over previous
import functools

import jax
import jax.numpy as jnp
from jax import lax
from jax.experimental import pallas as pl
from jax.experimental.pallas import tpu as pltpu
from jax.experimental.pallas import tpu_sc as plsc

N_NODES = 10000
N_EDGES = 320000
NC, NS = 2, 16
NW = NC * NS
CHUNK = 128
EDGES_PER_TILE = 10240
NCHUNK = EDGES_PER_TILE // CHUNK
ACC_ROWS = 10240
ROWS_PER_TILE = ACC_ROWS // NS
DUMMY = N_NODES
DEG_W = 16


def _mesh():
    return plsc.VectorSubcoreMesh(
        core_axis_name="c", subcore_axis_name="s", num_cores=NC, num_subcores=NS
    )


def _deg_body(dst_hbm, zeros_hbm, ones_hbm, out_hbm, dstv, onesv, acc):
    c = lax.axis_index("c")
    s = lax.axis_index("s")
    w = c * NS + s
    pltpu.sync_copy(dst_hbm.at[w], dstv)
    pltpu.sync_copy(ones_hbm, onesv)
    pltpu.sync_copy(zeros_hbm, acc.at[pl.ds(s * ROWS_PER_TILE, ROWS_PER_TILE)])
    plsc.subcore_barrier()

    def step(j, carry):
        pltpu.sync_copy(onesv, acc.at[dstv.at[j]], add=True)
        return carry

    lax.fori_loop(0, NCHUNK, step, 0)
    plsc.subcore_barrier()
    pltpu.sync_copy(
        acc.at[pl.ds(s * ROWS_PER_TILE, ROWS_PER_TILE)], out_hbm.at[w]
    )


def _scat_body(g_hbm, src_hbm, dst_hbm, zeros_hbm, out_hbm, srcv, dstv, buf, acc, sem):
    c = lax.axis_index("c")
    s = lax.axis_index("s")
    w = c * NS + s
    pltpu.sync_copy(src_hbm.at[w], srcv)
    pltpu.sync_copy(dst_hbm.at[w], dstv)
    pltpu.sync_copy(zeros_hbm, acc.at[pl.ds(s * ROWS_PER_TILE, ROWS_PER_TILE)])
    plsc.subcore_barrier()

    def step(j, carry):
        pltpu.async_copy(g_hbm.at[srcv.at[j]], buf, sem).wait()
        pltpu.sync_copy(buf, acc.at[dstv.at[j]], add=True)
        return carry

    lax.fori_loop(0, NCHUNK, step, 0)
    plsc.subcore_barrier()
    pltpu.sync_copy(
        acc.at[pl.ds(s * ROWS_PER_TILE, ROWS_PER_TILE)], out_hbm.at[w]
    )


def _deg_call(dst_r, zeros_d, ones_d):
    k = pl.kernel(
        _deg_body,
        out_type=jax.ShapeDtypeStruct((NW, ROWS_PER_TILE, DEG_W), jnp.float32),
        mesh=_mesh(),
        scratch_types=[
            pltpu.VMEM((NCHUNK, CHUNK), jnp.int32),
            pltpu.VMEM((CHUNK, DEG_W), jnp.float32),
            pltpu.VMEM_SHARED((ACC_ROWS, DEG_W), jnp.float32),
        ],
        compiler_params=pltpu.CompilerParams(use_tc_tiling_on_sc=False),
    )
    return k(dst_r, zeros_d, ones_d)


def _scat_call(g, src_r, dst_r, zeros64):
    k = pl.kernel(
        _scat_body,
        out_type=jax.ShapeDtypeStruct((NW, ROWS_PER_TILE, 64), jnp.float32),
        mesh=_mesh(),
        scratch_types=[
            pltpu.VMEM((NCHUNK, CHUNK), jnp.int32),
            pltpu.VMEM((NCHUNK, CHUNK), jnp.int32),
            pltpu.VMEM((CHUNK, 64), jnp.float32),
            pltpu.VMEM_SHARED((ACC_ROWS, 64), jnp.float32),
            pltpu.SemaphoreType.DMA,
        ],
        compiler_params=pltpu.CompilerParams(use_tc_tiling_on_sc=False),
    )
    return k(g, src_r, dst_r, zeros64)


def _l1_body(x_ref, w_ref, degp_ref, g_ref, dinv_ref):
    deg = (
        degp_ref[0, : N_NODES, 0:1]
        + degp_ref[1, : N_NODES, 0:1]
        + 1.0
    )
    dinv = lax.rsqrt(deg)
    xw = jnp.dot(x_ref[...], w_ref[...], preferred_element_type=jnp.float32)
    g_ref[...] = xw * dinv
    dinv_ref[...] = dinv


def _mid_body(accp_ref, gprev_ref, dinv_ref, b_ref, w_ref, g_ref):
    dinv = dinv_ref[...]
    z = (accp_ref[0, : N_NODES] + accp_ref[1, : N_NODES] + gprev_ref[...]) * dinv
    r = jnp.maximum(z + b_ref[...], 0.0)
    g_ref[...] = jnp.dot(r, w_ref[...], preferred_element_type=jnp.float32) * dinv


def _fin_body(accp_ref, gprev_ref, dinv_ref, b_ref, out_ref):
    z = (accp_ref[0, : N_NODES] + accp_ref[1, : N_NODES] + gprev_ref[...]) * dinv_ref[...]
    out_ref[...] = z + b_ref[...]


def _tc(body, out_shapes):
    return pl.pallas_call(body, out_shape=out_shapes)


def kernel(x, edge_index, W1, b1, W2, b2, W3, b3):
    src = edge_index[0]
    dst = edge_index[1]
    pad = NW * EDGES_PER_TILE - N_EDGES
    src_r = jnp.concatenate([src, jnp.zeros((pad,), jnp.int32)]).reshape(
        NW, NCHUNK, CHUNK
    )
    dst_r = jnp.concatenate([dst, jnp.full((pad,), DUMMY, jnp.int32)]).reshape(
        NW, NCHUNK, CHUNK
    )
    zeros64 = jnp.zeros((ROWS_PER_TILE, 64), jnp.float32)
    zeros_d = jnp.zeros((ROWS_PER_TILE, DEG_W), jnp.float32)
    ones_d = jnp.ones((CHUNK, DEG_W), jnp.float32)

    degp = _deg_call(dst_r, zeros_d, ones_d).reshape(NC, ACC_ROWS, DEG_W)

    g1, dinv = _tc(
        _l1_body,
        (
            jax.ShapeDtypeStruct((N_NODES, 64), jnp.float32),
            jax.ShapeDtypeStruct((N_NODES, 1), jnp.float32),
        ),
    )(x, W1, degp)

    a1 = _scat_call(g1, src_r, dst_r, zeros64).reshape(NC, ACC_ROWS, 64)
    g2 = _tc(_mid_body, jax.ShapeDtypeStruct((N_NODES, 64), jnp.float32))(
        a1, g1, dinv, b1.reshape(1, 64), W2
    )

    a2 = _scat_call(g2, src_r, dst_r, zeros64).reshape(NC, ACC_ROWS, 64)
    g3 = _tc(_mid_body, jax.ShapeDtypeStruct((N_NODES, 64), jnp.float32))(
        a2, g2, dinv, b2.reshape(1, 64), W3
    )

    a3 = _scat_call(g3, src_r, dst_r, zeros64).reshape(NC, ACC_ROWS, 64)
    out = _tc(_fin_body, jax.ShapeDtypeStruct((N_NODES, 64), jnp.float32))(
        a3, g3, dinv, b3.reshape(1, 64)
    )
    return out


# speedup vs baseline: 13.3456x; 1.1049x over previous
import functools

import jax
import jax.numpy as jnp
from jax import lax
from jax.experimental import pallas as pl
from jax.experimental.pallas import tpu as pltpu
from jax.experimental.pallas import tpu_sc as plsc

N_NODES = 10000
N_EDGES = 320000
NC, NS = 2, 16
NW = NC * NS
CHUNK = 512
NSTREAM = 20
EDGES_PER_TILE = NSTREAM * CHUNK
ACC_ROWS = 10240
ROWS_PER_TILE = ACC_ROWS // NS
DUMMY = N_NODES
DEG_W = 16


def _mesh():
    return plsc.VectorSubcoreMesh(
        core_axis_name="c", subcore_axis_name="s", num_cores=NC, num_subcores=NS
    )


def _deg_body(dst_hbm, zeros_hbm, ones_hbm, out_hbm, dstv, onesv, acc):
    c = lax.axis_index("c")
    s = lax.axis_index("s")
    w = c * NS + s
    pltpu.sync_copy(dst_hbm.at[w], dstv)
    pltpu.sync_copy(ones_hbm, onesv)
    pltpu.sync_copy(zeros_hbm, acc.at[pl.ds(s * ROWS_PER_TILE, ROWS_PER_TILE)])
    plsc.subcore_barrier()

    def step(j, carry):
        pltpu.sync_copy(onesv, acc.at[dstv.at[j]], add=True)
        return carry

    lax.fori_loop(0, NSTREAM, step, 0)
    plsc.subcore_barrier()
    pltpu.sync_copy(
        acc.at[pl.ds(s * ROWS_PER_TILE, ROWS_PER_TILE)], out_hbm.at[w]
    )


def _scat_body(
    g_hbm, src_hbm, dst_hbm, zeros_hbm, out_hbm,
    srcv, dstv, buf0, buf1, acc, gsem0, gsem1, ssem0, ssem1,
):
    c = lax.axis_index("c")
    s = lax.axis_index("s")
    w = c * NS + s
    pltpu.sync_copy(src_hbm.at[w], srcv)
    pltpu.sync_copy(dst_hbm.at[w], dstv)
    pltpu.sync_copy(zeros_hbm, acc.at[pl.ds(s * ROWS_PER_TILE, ROWS_PER_TILE)])
    plsc.subcore_barrier()

    bufs = (buf0, buf1)
    gsems = (gsem0, gsem1)
    ssems = (ssem0, ssem1)

    def gather(j, b):
        pltpu.async_copy(
            g_hbm.at[srcv.at[j]], bufs[b], gsems[b]
        )

    def wait_gather(b):
        pltpu.make_async_copy(
            g_hbm.at[srcv.at[0]], bufs[b], gsems[b]
        ).wait()

    def scat(j, b):
        pltpu.async_copy(
            bufs[b], acc.at[dstv.at[j]], ssems[b], add=True
        )

    def wait_scat(b):
        pltpu.make_async_copy(
            bufs[b], acc.at[dstv.at[0]], ssems[b]
        ).wait()

    gather(0, 0)

    def step(jo, carry):
        for b in range(2):
            j = jo * 2 + b
            wait_gather(b)

            @pl.when(j + 1 < NSTREAM)
            def _():
                @pl.when(j >= 1)
                def _():
                    wait_scat(1 - b)

                gather(j + 1, 1 - b)

            scat(j, b)
        return carry

    lax.fori_loop(0, NSTREAM // 2, step, 0)
    wait_scat(0)
    wait_scat(1)
    plsc.subcore_barrier()
    pltpu.sync_copy(
        acc.at[pl.ds(s * ROWS_PER_TILE, ROWS_PER_TILE)], out_hbm.at[w]
    )


def _deg_call(dst_r, zeros_d, ones_d):
    k = pl.kernel(
        _deg_body,
        out_type=jax.ShapeDtypeStruct((NW, ROWS_PER_TILE, DEG_W), jnp.float32),
        mesh=_mesh(),
        scratch_types=[
            pltpu.VMEM((NSTREAM, CHUNK), jnp.int32),
            pltpu.VMEM((CHUNK, DEG_W), jnp.float32),
            pltpu.VMEM_SHARED((ACC_ROWS, DEG_W), jnp.float32),
        ],
        compiler_params=pltpu.CompilerParams(use_tc_tiling_on_sc=False),
    )
    return k(dst_r, zeros_d, ones_d)


def _scat_call(g, src_r, dst_r, zeros64):
    k = pl.kernel(
        _scat_body,
        out_type=jax.ShapeDtypeStruct((NW, ROWS_PER_TILE, 64), jnp.float32),
        mesh=_mesh(),
        scratch_types=[
            pltpu.VMEM((NSTREAM, CHUNK), jnp.int32),
            pltpu.VMEM((NSTREAM, CHUNK), jnp.int32),
            pltpu.VMEM((CHUNK, 64), jnp.float32),
            pltpu.VMEM((CHUNK, 64), jnp.float32),
            pltpu.VMEM_SHARED((ACC_ROWS, 64), jnp.float32),
            pltpu.SemaphoreType.DMA,
            pltpu.SemaphoreType.DMA,
            pltpu.SemaphoreType.DMA,
            pltpu.SemaphoreType.DMA,
        ],
        compiler_params=pltpu.CompilerParams(use_tc_tiling_on_sc=False),
    )
    return k(g, src_r, dst_r, zeros64)


def _l1_body(x_ref, w_ref, degp_ref, g_ref, dinv_ref):
    deg = (
        degp_ref[0, : N_NODES, 0:1]
        + degp_ref[1, : N_NODES, 0:1]
        + 1.0
    )
    dinv = lax.rsqrt(deg)
    xw = jnp.dot(x_ref[...], w_ref[...], preferred_element_type=jnp.float32)
    g_ref[...] = xw * dinv
    dinv_ref[...] = dinv


def _mid_body(accp_ref, gprev_ref, dinv_ref, b_ref, w_ref, g_ref):
    dinv = dinv_ref[...]
    z = (accp_ref[0, : N_NODES] + accp_ref[1, : N_NODES] + gprev_ref[...]) * dinv
    r = jnp.maximum(z + b_ref[...], 0.0)
    g_ref[...] = jnp.dot(r, w_ref[...], preferred_element_type=jnp.float32) * dinv


def _fin_body(accp_ref, gprev_ref, dinv_ref, b_ref, out_ref):
    z = (accp_ref[0, : N_NODES] + accp_ref[1, : N_NODES] + gprev_ref[...]) * dinv_ref[...]
    out_ref[...] = z + b_ref[...]


def _tc(body, out_shapes):
    return pl.pallas_call(body, out_shape=out_shapes)


def kernel(x, edge_index, W1, b1, W2, b2, W3, b3):
    src = edge_index[0]
    dst = edge_index[1]
    pad = NW * EDGES_PER_TILE - N_EDGES
    src_r = jnp.concatenate([src, jnp.zeros((pad,), jnp.int32)]).reshape(
        NW, NSTREAM, CHUNK
    )
    dst_r = jnp.concatenate([dst, jnp.full((pad,), DUMMY, jnp.int32)]).reshape(
        NW, NSTREAM, CHUNK
    )
    zeros64 = jnp.zeros((ROWS_PER_TILE, 64), jnp.float32)
    zeros_d = jnp.zeros((ROWS_PER_TILE, DEG_W), jnp.float32)
    ones_d = jnp.ones((CHUNK, DEG_W), jnp.float32)

    degp = _deg_call(dst_r, zeros_d, ones_d).reshape(NC, ACC_ROWS, DEG_W)

    g1, dinv = _tc(
        _l1_body,
        (
            jax.ShapeDtypeStruct((N_NODES, 64), jnp.float32),
            jax.ShapeDtypeStruct((N_NODES, 1), jnp.float32),
        ),
    )(x, W1, degp)

    a1 = _scat_call(g1, src_r, dst_r, zeros64).reshape(NC, ACC_ROWS, 64)
    g2 = _tc(_mid_body, jax.ShapeDtypeStruct((N_NODES, 64), jnp.float32))(
        a1, g1, dinv, b1.reshape(1, 64), W2
    )

    a2 = _scat_call(g2, src_r, dst_r, zeros64).reshape(NC, ACC_ROWS, 64)
    g3 = _tc(_mid_body, jax.ShapeDtypeStruct((N_NODES, 64), jnp.float32))(
        a2, g2, dinv, b2.reshape(1, 64), W3
    )

    a3 = _scat_call(g3, src_r, dst_r, zeros64).reshape(NC, ACC_ROWS, 64)
    out = _tc(_fin_body, jax.ShapeDtypeStruct((N_NODES, 64), jnp.float32))(
        a3, g3, dinv, b3.reshape(1, 64)
    )
    return out


# speedup vs baseline: 14.4554x; 1.0832x over previous
import functools

import jax
import jax.numpy as jnp
from jax import lax
from jax.experimental import pallas as pl
from jax.experimental.pallas import tpu as pltpu
from jax.experimental.pallas import tpu_sc as plsc

N_NODES = 10000
N_EDGES = 320000
NC, NS = 2, 16
NW = NC * NS
CHUNK = 512
NSTREAM = 20
EDGES_PER_TILE = NSTREAM * CHUNK
ACC_ROWS = 10240
ROWS_PER_TILE = ACC_ROWS // NS
DUMMY = N_NODES
DEG_W = 16


def _mesh():
    return plsc.VectorSubcoreMesh(
        core_axis_name="c", subcore_axis_name="s", num_cores=NC, num_subcores=NS
    )


def _deg_body(dst_hbm, zeros_hbm, ones_hbm, out_hbm, dstv, onesv, acc):
    c = lax.axis_index("c")
    s = lax.axis_index("s")
    w = c * NS + s
    pltpu.sync_copy(dst_hbm.at[w], dstv)
    pltpu.sync_copy(ones_hbm, onesv)
    pltpu.sync_copy(zeros_hbm, acc.at[pl.ds(s * ROWS_PER_TILE, ROWS_PER_TILE)])
    plsc.subcore_barrier()

    def step(j, carry):
        pltpu.sync_copy(onesv, acc.at[dstv.at[j]], add=True)
        return carry

    lax.fori_loop(0, NSTREAM, step, 0)
    plsc.subcore_barrier()
    pltpu.sync_copy(
        acc.at[pl.ds(s * ROWS_PER_TILE, ROWS_PER_TILE)], out_hbm.at[w]
    )


def _scat_body(
    g_hbm, src_hbm, dst_hbm, zeros_hbm, out_hbm,
    srcv, dstv, buf0, buf1, acc, gsem0, gsem1, ssem0, ssem1,
):
    c = lax.axis_index("c")
    s = lax.axis_index("s")
    w = c * NS + s
    pltpu.sync_copy(src_hbm.at[w], srcv)
    pltpu.sync_copy(dst_hbm.at[w], dstv)
    pltpu.sync_copy(zeros_hbm, acc.at[pl.ds(s * ROWS_PER_TILE, ROWS_PER_TILE)])
    plsc.subcore_barrier()

    bufs = (buf0, buf1)
    gsems = (gsem0, gsem1)
    ssems = (ssem0, ssem1)

    def gather(j, b):
        pltpu.async_copy(
            g_hbm.at[srcv.at[j]], bufs[b], gsems[b]
        )

    def wait_gather(b):
        pltpu.make_async_copy(
            g_hbm.at[srcv.at[0]], bufs[b], gsems[b]
        ).wait()

    def scat(j, b):
        pltpu.async_copy(
            bufs[b], acc.at[dstv.at[j]], ssems[b], add=True
        )

    def wait_scat(b):
        pltpu.make_async_copy(
            bufs[b], acc.at[dstv.at[0]], ssems[b]
        ).wait()

    gather(0, 0)

    def step(jo, carry):
        for b in range(2):
            j = jo * 2 + b
            wait_gather(b)

            @pl.when(j + 1 < NSTREAM)
            def _():
                @pl.when(j >= 1)
                def _():
                    wait_scat(1 - b)

                gather(j + 1, 1 - b)

            scat(j, b)
        return carry

    lax.fori_loop(0, NSTREAM // 2, step, 0)
    wait_scat(0)
    wait_scat(1)
    plsc.subcore_barrier()
    pltpu.sync_copy(
        acc.at[pl.ds(s * ROWS_PER_TILE, ROWS_PER_TILE)], out_hbm.at[w]
    )


def _deg_call(dst_r, zeros_d, ones_d):
    k = pl.kernel(
        _deg_body,
        out_type=jax.ShapeDtypeStruct((NW, ROWS_PER_TILE, DEG_W), jnp.float32),
        mesh=_mesh(),
        scratch_types=[
            pltpu.VMEM((NSTREAM, CHUNK), jnp.int32),
            pltpu.VMEM((CHUNK, DEG_W), jnp.float32),
            pltpu.VMEM_SHARED((ACC_ROWS, DEG_W), jnp.float32),
        ],
        compiler_params=pltpu.CompilerParams(use_tc_tiling_on_sc=False),
    )
    return k(dst_r, zeros_d, ones_d)


def _scat_call(g, src_r, dst_r, zeros64):
    k = pl.kernel(
        _scat_body,
        out_type=jax.ShapeDtypeStruct((NW, ROWS_PER_TILE, 64), jnp.float32),
        mesh=_mesh(),
        scratch_types=[
            pltpu.VMEM((NSTREAM, CHUNK), jnp.int32),
            pltpu.VMEM((NSTREAM, CHUNK), jnp.int32),
            pltpu.VMEM((CHUNK, 64), jnp.float32),
            pltpu.VMEM((CHUNK, 64), jnp.float32),
            pltpu.VMEM_SHARED((ACC_ROWS, 64), jnp.float32),
            pltpu.SemaphoreType.DMA,
            pltpu.SemaphoreType.DMA,
            pltpu.SemaphoreType.DMA,
            pltpu.SemaphoreType.DMA,
        ],
        compiler_params=pltpu.CompilerParams(use_tc_tiling_on_sc=False),
    )
    return k(g, src_r, dst_r, zeros64)


def _l1_body(x_ref, w_ref, degp_ref, g_ref, dinv_ref):
    deg = (
        degp_ref[0, : N_NODES, 0:1]
        + degp_ref[1, : N_NODES, 0:1]
        + 1.0
    )
    dinv = lax.rsqrt(deg)
    xw = jnp.dot(x_ref[...], w_ref[...], preferred_element_type=jnp.float32)
    g_ref[...] = xw * dinv
    dinv_ref[...] = dinv


def _mid_body(accp_ref, gprev_ref, dinv_ref, b_ref, w_ref, g_ref):
    dinv = dinv_ref[...]
    z = (accp_ref[0, : N_NODES] + accp_ref[1, : N_NODES] + gprev_ref[...]) * dinv
    r = jnp.maximum(z + b_ref[...], 0.0)
    g_ref[...] = jnp.dot(r, w_ref[...], preferred_element_type=jnp.float32) * dinv


def _fin_body(accp_ref, gprev_ref, dinv_ref, b_ref, out_ref):
    z = (accp_ref[0, : N_NODES] + accp_ref[1, : N_NODES] + gprev_ref[...]) * dinv_ref[...]
    out_ref[...] = z + b_ref[...]


def _tc(body, out_shapes):
    return pl.pallas_call(body, out_shape=out_shapes)


def kernel(x, edge_index, W1, b1, W2, b2, W3, b3):
    src = edge_index[0]
    dst = edge_index[1]
    pad = NW * EDGES_PER_TILE - N_EDGES
    src_r = jnp.concatenate([src, jnp.zeros((pad,), jnp.int32)]).reshape(
        NW, NSTREAM, CHUNK
    )
    pad_dst = DUMMY + jnp.arange(pad, dtype=jnp.int32) % (ACC_ROWS - N_NODES)
    dst_r = jnp.concatenate([dst, pad_dst]).reshape(NW, NSTREAM, CHUNK)
    zeros64 = jnp.zeros((ROWS_PER_TILE, 64), jnp.float32)
    zeros_d = jnp.zeros((ROWS_PER_TILE, DEG_W), jnp.float32)
    ones_d = jnp.ones((CHUNK, DEG_W), jnp.float32)

    degp = _deg_call(dst_r, zeros_d, ones_d).reshape(NC, ACC_ROWS, DEG_W)

    g1, dinv = _tc(
        _l1_body,
        (
            jax.ShapeDtypeStruct((N_NODES, 64), jnp.float32),
            jax.ShapeDtypeStruct((N_NODES, 1), jnp.float32),
        ),
    )(x, W1, degp)

    a1 = _scat_call(g1, src_r, dst_r, zeros64).reshape(NC, ACC_ROWS, 64)
    g2 = _tc(_mid_body, jax.ShapeDtypeStruct((N_NODES, 64), jnp.float32))(
        a1, g1, dinv, b1.reshape(1, 64), W2
    )

    a2 = _scat_call(g2, src_r, dst_r, zeros64).reshape(NC, ACC_ROWS, 64)
    g3 = _tc(_mid_body, jax.ShapeDtypeStruct((N_NODES, 64), jnp.float32))(
        a2, g2, dinv, b2.reshape(1, 64), W3
    )

    a3 = _scat_call(g3, src_r, dst_r, zeros64).reshape(NC, ACC_ROWS, 64)
    out = _tc(_fin_body, jax.ShapeDtypeStruct((N_NODES, 64), jnp.float32))(
        a3, g3, dinv, b3.reshape(1, 64)
    )
    return out


# speedup vs baseline: 14.5118x; 1.0039x over previous
import jax
import jax.numpy as jnp
from jax import lax
from jax.experimental import pallas as pl
from jax.experimental.pallas import tpu as pltpu
from jax.experimental.pallas import tpu_sc as plsc

N_NODES = 10000
N_EDGES = 320000
NC, NS = 2, 16
NW = NC * NS
CHUNK = 256
NSTREAM = 40
NBUF = 4
EDGES_PER_TILE = NSTREAM * CHUNK
ACC_ROWS = 10240
ROWS_PER_TILE = ACC_ROWS // NS
DUMMY = N_NODES
DEG_W = 16


def _mesh():
    return plsc.VectorSubcoreMesh(
        core_axis_name="c", subcore_axis_name="s", num_cores=NC, num_subcores=NS
    )


def _deg_body(dst_hbm, zeros_hbm, ones_hbm, out_hbm, dstv, onesv, acc):
    c = lax.axis_index("c")
    s = lax.axis_index("s")
    w = c * NS + s
    pltpu.sync_copy(dst_hbm.at[w], dstv)
    pltpu.sync_copy(ones_hbm, onesv)
    pltpu.sync_copy(zeros_hbm, acc.at[pl.ds(s * ROWS_PER_TILE, ROWS_PER_TILE)])
    plsc.subcore_barrier()

    def step(j, carry):
        pltpu.sync_copy(onesv, acc.at[dstv.at[j]], add=True)
        return carry

    lax.fori_loop(0, NSTREAM, step, 0)
    plsc.subcore_barrier()
    pltpu.sync_copy(
        acc.at[pl.ds(s * ROWS_PER_TILE, ROWS_PER_TILE)], out_hbm.at[w]
    )


def _scat_body(
    g_hbm, src_hbm, dst_hbm, zeros_hbm, out_hbm,
    srcv, dstv, bufs, acc, gsems, ssems,
):
    c = lax.axis_index("c")
    s = lax.axis_index("s")
    w = c * NS + s
    pltpu.sync_copy(src_hbm.at[w], srcv)
    pltpu.sync_copy(dst_hbm.at[w], dstv)
    pltpu.sync_copy(zeros_hbm, acc.at[pl.ds(s * ROWS_PER_TILE, ROWS_PER_TILE)])
    plsc.subcore_barrier()

    def gather(j, b):
        pltpu.async_copy(g_hbm.at[srcv.at[j]], bufs[b], gsems[b])

    def wait_gather(b):
        pltpu.make_async_copy(g_hbm.at[srcv.at[0]], bufs[b], gsems[b]).wait()

    def scat(j, b):
        pltpu.async_copy(bufs[b], acc.at[dstv.at[j]], ssems[b], add=True)

    def wait_scat(b):
        pltpu.make_async_copy(bufs[b], acc.at[dstv.at[0]], ssems[b]).wait()

    for k in range(NBUF - 1):
        gather(k, k)

    def step(jo, carry):
        for db in range(NBUF):
            j = jo * NBUF + db
            wait_gather(db)
            jn = j + (NBUF - 1)
            bn = (db + NBUF - 1) % NBUF

            @pl.when(jn < NSTREAM)
            def _():
                @pl.when(jn >= NBUF)
                def _():
                    wait_scat(bn)

                gather(jn, bn)

            scat(j, db)
        return carry

    lax.fori_loop(0, NSTREAM // NBUF, step, 0)
    for b in range(NBUF):
        wait_scat(b)
    plsc.subcore_barrier()
    pltpu.sync_copy(
        acc.at[pl.ds(s * ROWS_PER_TILE, ROWS_PER_TILE)], out_hbm.at[w]
    )


def _scat_body_wrap(
    g_hbm, src_hbm, dst_hbm, zeros_hbm, out_hbm,
    srcv, dstv, b0, b1, b2, b3, acc, g0, g1, g2, g3, s0, s1, s2, s3,
):
    _scat_body(
        g_hbm, src_hbm, dst_hbm, zeros_hbm, out_hbm,
        srcv, dstv, (b0, b1, b2, b3), acc, (g0, g1, g2, g3), (s0, s1, s2, s3),
    )


def _deg_call(dst_r, zeros_d, ones_d):
    k = pl.kernel(
        _deg_body,
        out_type=jax.ShapeDtypeStruct((NW, ROWS_PER_TILE, DEG_W), jnp.float32),
        mesh=_mesh(),
        scratch_types=[
            pltpu.VMEM((NSTREAM, CHUNK), jnp.int32),
            pltpu.VMEM((CHUNK, DEG_W), jnp.float32),
            pltpu.VMEM_SHARED((ACC_ROWS, DEG_W), jnp.float32),
        ],
        compiler_params=pltpu.CompilerParams(use_tc_tiling_on_sc=False),
    )
    return k(dst_r, zeros_d, ones_d)


def _scat_call(g, src_r, dst_r, zeros64):
    k = pl.kernel(
        _scat_body_wrap,
        out_type=jax.ShapeDtypeStruct((NW, ROWS_PER_TILE, 64), jnp.float32),
        mesh=_mesh(),
        scratch_types=[
            pltpu.VMEM((NSTREAM, CHUNK), jnp.int32),
            pltpu.VMEM((NSTREAM, CHUNK), jnp.int32),
            pltpu.VMEM((CHUNK, 64), jnp.float32),
            pltpu.VMEM((CHUNK, 64), jnp.float32),
            pltpu.VMEM((CHUNK, 64), jnp.float32),
            pltpu.VMEM((CHUNK, 64), jnp.float32),
            pltpu.VMEM_SHARED((ACC_ROWS, 64), jnp.float32),
            pltpu.SemaphoreType.DMA,
            pltpu.SemaphoreType.DMA,
            pltpu.SemaphoreType.DMA,
            pltpu.SemaphoreType.DMA,
            pltpu.SemaphoreType.DMA,
            pltpu.SemaphoreType.DMA,
            pltpu.SemaphoreType.DMA,
            pltpu.SemaphoreType.DMA,
        ],
        compiler_params=pltpu.CompilerParams(use_tc_tiling_on_sc=False),
    )
    return k(g, src_r, dst_r, zeros64)


def _l1_body(x_ref, w_ref, degp_ref, g_ref, dinv_ref):
    deg = (
        degp_ref[0, : N_NODES, 0:1]
        + degp_ref[1, : N_NODES, 0:1]
        + 1.0
    )
    dinv = lax.rsqrt(deg)
    xw = jnp.dot(x_ref[...], w_ref[...], preferred_element_type=jnp.float32)
    g_ref[...] = xw * dinv
    dinv_ref[...] = dinv


def _mid_body(accp_ref, gprev_ref, dinv_ref, b_ref, w_ref, g_ref):
    dinv = dinv_ref[...]
    z = (accp_ref[0, : N_NODES] + accp_ref[1, : N_NODES] + gprev_ref[...]) * dinv
    r = jnp.maximum(z + b_ref[...], 0.0)
    g_ref[...] = jnp.dot(r, w_ref[...], preferred_element_type=jnp.float32) * dinv


def _fin_body(accp_ref, gprev_ref, dinv_ref, b_ref, out_ref):
    z = (accp_ref[0, : N_NODES] + accp_ref[1, : N_NODES] + gprev_ref[...]) * dinv_ref[...]
    out_ref[...] = z + b_ref[...]


def _tc(body, out_shapes):
    return pl.pallas_call(body, out_shape=out_shapes)


def kernel(x, edge_index, W1, b1, W2, b2, W3, b3):
    src = edge_index[0]
    dst = edge_index[1]
    pad = NW * EDGES_PER_TILE - N_EDGES
    src_r = jnp.concatenate([src, jnp.zeros((pad,), jnp.int32)]).reshape(
        NW, NSTREAM, CHUNK
    )
    pad_dst = DUMMY + jnp.arange(pad, dtype=jnp.int32) % (ACC_ROWS - N_NODES)
    dst_r = jnp.concatenate([dst, pad_dst]).reshape(NW, NSTREAM, CHUNK)
    zeros64 = jnp.zeros((ROWS_PER_TILE, 64), jnp.float32)
    zeros_d = jnp.zeros((ROWS_PER_TILE, DEG_W), jnp.float32)
    ones_d = jnp.ones((CHUNK, DEG_W), jnp.float32)

    degp = _deg_call(dst_r, zeros_d, ones_d).reshape(NC, ACC_ROWS, DEG_W)

    g1, dinv = _tc(
        _l1_body,
        (
            jax.ShapeDtypeStruct((N_NODES, 64), jnp.float32),
            jax.ShapeDtypeStruct((N_NODES, 1), jnp.float32),
        ),
    )(x, W1, degp)

    a1 = _scat_call(g1, src_r, dst_r, zeros64).reshape(NC, ACC_ROWS, 64)
    g2 = _tc(_mid_body, jax.ShapeDtypeStruct((N_NODES, 64), jnp.float32))(
        a1, g1, dinv, b1.reshape(1, 64), W2
    )

    a2 = _scat_call(g2, src_r, dst_r, zeros64).reshape(NC, ACC_ROWS, 64)
    g3 = _tc(_mid_body, jax.ShapeDtypeStruct((N_NODES, 64), jnp.float32))(
        a2, g2, dinv, b2.reshape(1, 64), W3
    )

    a3 = _scat_call(g3, src_r, dst_r, zeros64).reshape(NC, ACC_ROWS, 64)
    out = _tc(_fin_body, jax.ShapeDtypeStruct((N_NODES, 64), jnp.float32))(
        a3, g3, dinv, b3.reshape(1, 64)
    )
    return out


# speedup vs baseline: 15.6528x; 1.0786x over previous
import jax
import jax.numpy as jnp
from jax import lax
from jax.experimental import pallas as pl
from jax.experimental.pallas import tpu as pltpu
from jax.experimental.pallas import tpu_sc as plsc

N_NODES = 10000
N_EDGES = 320000
NC, NS = 2, 16
NW = NC * NS
CHUNK = 256
NS0 = 40
NS1 = 40
NBUF = 4
NSTREAMS_TOTAL = NS * (NS0 + NS1)
TOTAL_EDGES = NSTREAMS_TOTAL * CHUNK
ACC_ROWS = 10240
ROWS_PER_TILE = ACC_ROWS // NS
DUMMY = N_NODES
DEG_W = 4


def _mesh():
    return plsc.VectorSubcoreMesh(
        core_axis_name="c", subcore_axis_name="s", num_cores=NC, num_subcores=NS
    )


def _load_idx(c, s, a_hbm, av):
    w = c * NS + s
    pltpu.sync_copy(a_hbm.at[pl.ds(w * NS0, NS0)], av)


def _deg_body(dst_hbm, zeros_hbm, ones_hbm, out_hbm, dstv, onesv, acc):
    c = lax.axis_index("c")
    s = lax.axis_index("s")
    w = c * NS + s
    _load_idx(c, s, dst_hbm, dstv)
    pltpu.sync_copy(ones_hbm, onesv)
    pltpu.sync_copy(zeros_hbm, acc.at[pl.ds(s * ROWS_PER_TILE, ROWS_PER_TILE)])
    plsc.subcore_barrier()

    def step(j, carry):
        pltpu.sync_copy(onesv, acc.at[dstv.at[j]], add=True)
        return carry

    lax.fori_loop(0, NS0, step, 0)
    plsc.subcore_barrier()
    pltpu.sync_copy(
        acc.at[pl.ds(s * ROWS_PER_TILE, ROWS_PER_TILE)], out_hbm.at[w]
    )


def _scat_body(
    g_hbm, src_hbm, dst_hbm, zeros_hbm, out_hbm,
    srcv, dstv, bufs, acc, gsems, ssems,
):
    c = lax.axis_index("c")
    s = lax.axis_index("s")
    w = c * NS + s
    _load_idx(c, s, src_hbm, srcv)
    _load_idx(c, s, dst_hbm, dstv)
    pltpu.sync_copy(zeros_hbm, acc.at[pl.ds(s * ROWS_PER_TILE, ROWS_PER_TILE)])
    plsc.subcore_barrier()

    def gather(j, b):
        pltpu.async_copy(g_hbm.at[srcv.at[j]], bufs[b], gsems[b])

    def wait_gather(b):
        pltpu.make_async_copy(g_hbm.at[srcv.at[0]], bufs[b], gsems[b]).wait()

    def scat(j, b):
        pltpu.async_copy(bufs[b], acc.at[dstv.at[j]], ssems[b], add=True)

    def wait_scat(b):
        pltpu.make_async_copy(bufs[b], acc.at[dstv.at[0]], ssems[b]).wait()

    def pipeline(n):
        for k in range(NBUF - 1):
            gather(k, k)

        def step(jo, carry):
            for db in range(NBUF):
                j = jo * NBUF + db
                wait_gather(db)
                jn = j + (NBUF - 1)
                bn = (db + NBUF - 1) % NBUF

                @pl.when(jn < n)
                def _():
                    @pl.when(jn >= NBUF)
                    def _():
                        wait_scat(bn)

                    gather(jn, bn)

                scat(j, db)
            return carry

        lax.fori_loop(0, n // NBUF, step, 0)
        for b in range(NBUF):
            wait_scat(b)

    pipeline(NS0)
    plsc.subcore_barrier()
    pltpu.sync_copy(
        acc.at[pl.ds(s * ROWS_PER_TILE, ROWS_PER_TILE)], out_hbm.at[w]
    )


def _scat_body_wrap(
    g_hbm, src_hbm, dst_hbm, zeros_hbm, out_hbm,
    srcv, dstv, b0, b1, b2, b3, acc, g0, g1, g2, g3, s0, s1, s2, s3,
):
    _scat_body(
        g_hbm, src_hbm, dst_hbm, zeros_hbm, out_hbm,
        srcv, dstv, (b0, b1, b2, b3), acc, (g0, g1, g2, g3), (s0, s1, s2, s3),
    )


def _deg_call(dst_r, zeros_d, ones_d):
    k = pl.kernel(
        _deg_body,
        out_type=jax.ShapeDtypeStruct((NW, ROWS_PER_TILE, DEG_W), jnp.float32),
        mesh=_mesh(),
        scratch_types=[
            pltpu.VMEM((NS0, CHUNK), jnp.int32),
            pltpu.VMEM((CHUNK, DEG_W), jnp.float32),
            pltpu.VMEM_SHARED((ACC_ROWS, DEG_W), jnp.float32),
        ],
        compiler_params=pltpu.CompilerParams(use_tc_tiling_on_sc=False),
    )
    return k(dst_r, zeros_d, ones_d)


def _scat_call(g, src_r, dst_r, zeros64):
    k = pl.kernel(
        _scat_body_wrap,
        out_type=jax.ShapeDtypeStruct((NW, ROWS_PER_TILE, 64), jnp.float32),
        mesh=_mesh(),
        scratch_types=[
            pltpu.VMEM((NS0, CHUNK), jnp.int32),
            pltpu.VMEM((NS0, CHUNK), jnp.int32),
            pltpu.VMEM((CHUNK, 64), jnp.float32),
            pltpu.VMEM((CHUNK, 64), jnp.float32),
            pltpu.VMEM((CHUNK, 64), jnp.float32),
            pltpu.VMEM((CHUNK, 64), jnp.float32),
            pltpu.VMEM_SHARED((ACC_ROWS, 64), jnp.float32),
            pltpu.SemaphoreType.DMA,
            pltpu.SemaphoreType.DMA,
            pltpu.SemaphoreType.DMA,
            pltpu.SemaphoreType.DMA,
            pltpu.SemaphoreType.DMA,
            pltpu.SemaphoreType.DMA,
            pltpu.SemaphoreType.DMA,
            pltpu.SemaphoreType.DMA,
        ],
        compiler_params=pltpu.CompilerParams(use_tc_tiling_on_sc=False),
    )
    return k(g, src_r, dst_r, zeros64)


def _l1_body(x_ref, w_ref, degp_ref, g_ref, dinv_ref):
    deg = (
        degp_ref[0, : N_NODES, 0:1]
        + degp_ref[1, : N_NODES, 0:1]
        + 1.0
    )
    dinv = lax.rsqrt(deg)
    xw = jnp.dot(x_ref[...], w_ref[...], preferred_element_type=jnp.float32)
    gv = xw * dinv
    g_ref[: N_NODES] = gv
    g_ref[N_NODES :] = gv
    dinv_ref[...] = dinv


def _mid_body(accp_ref, gprev_ref, dinv_ref, b_ref, w_ref, g_ref):
    dinv = dinv_ref[...]
    z = (accp_ref[0, : N_NODES] + accp_ref[1, : N_NODES] + gprev_ref[: N_NODES]) * dinv
    r = jnp.maximum(z + b_ref[...], 0.0)
    gv = jnp.dot(r, w_ref[...], preferred_element_type=jnp.float32) * dinv
    g_ref[: N_NODES] = gv
    g_ref[N_NODES :] = gv


def _fin_body(accp_ref, gprev_ref, dinv_ref, b_ref, out_ref):
    z = (accp_ref[0, : N_NODES] + accp_ref[1, : N_NODES] + gprev_ref[: N_NODES]) * dinv_ref[...]
    out_ref[...] = z + b_ref[...]


def _tc(body, out_shapes):
    return pl.pallas_call(body, out_shape=out_shapes)


def kernel(x, edge_index, W1, b1, W2, b2, W3, b3):
    src = edge_index[0]
    dst = edge_index[1]
    pad = TOTAL_EDGES - N_EDGES
    pad_dst = DUMMY + jnp.arange(pad, dtype=jnp.int32) % (ACC_ROWS - N_NODES)
    src_r = jnp.concatenate([src, jnp.zeros((pad,), jnp.int32)]).reshape(
        NSTREAMS_TOTAL, CHUNK
    )
    core_off = jnp.where(
        jnp.arange(NSTREAMS_TOTAL, dtype=jnp.int32)[:, None] >= NS * NS0,
        jnp.int32(N_NODES), jnp.int32(0),
    )
    src_r = src_r + core_off
    dst_r = jnp.concatenate([dst, pad_dst]).reshape(NSTREAMS_TOTAL, CHUNK)
    zeros64 = jnp.zeros((ROWS_PER_TILE, 64), jnp.float32)
    zeros_d = jnp.zeros((ROWS_PER_TILE, DEG_W), jnp.float32)
    ones_d = jnp.ones((CHUNK, DEG_W), jnp.float32)

    degp = _deg_call(dst_r, zeros_d, ones_d).reshape(NC, ACC_ROWS, DEG_W)

    g1, dinv = _tc(
        _l1_body,
        (
            jax.ShapeDtypeStruct((2 * N_NODES, 64), jnp.float32),
            jax.ShapeDtypeStruct((N_NODES, 1), jnp.float32),
        ),
    )(x, W1, degp)

    a1 = _scat_call(g1, src_r, dst_r, zeros64).reshape(NC, ACC_ROWS, 64)
    g2 = _tc(_mid_body, jax.ShapeDtypeStruct((2 * N_NODES, 64), jnp.float32))(
        a1, g1, dinv, b1.reshape(1, 64), W2
    )

    a2 = _scat_call(g2, src_r, dst_r, zeros64).reshape(NC, ACC_ROWS, 64)
    g3 = _tc(_mid_body, jax.ShapeDtypeStruct((2 * N_NODES, 64), jnp.float32))(
        a2, g2, dinv, b2.reshape(1, 64), W3
    )

    a3 = _scat_call(g3, src_r, dst_r, zeros64).reshape(NC, ACC_ROWS, 64)
    out = _tc(_fin_body, jax.ShapeDtypeStruct((N_NODES, 64), jnp.float32))(
        a3, g3, dinv, b3.reshape(1, 64)
    )
    return out
